# Initial kernel scaffold; baseline (speedup 1.0000x reference)
import jax
import jax.numpy as jnp
from jax.experimental import pallas as pl


def kernel(node_state, edge_state, node_dyn, edge_dyn, node_static, edge_static, edge_index, W_ne, b_ne, g_ne, be_ne, W_ee, b_ee, g_ee, be_ee, W_m1, b_m1, W_m2, b_m2, W_mn, b_mn, W_m3, b_m3, W_m4, b_m4, W_mn2, b_mn2, W_nu, b_nu, W_eu, b_eu, Wih1, Whh1, bih1, bhh1, Wih2, Whh2, bih2, bhh2, Wihe, Whhe, bihe, bhhe, g_nn, b_nn, g_en, b_en):
    raise NotImplementedError("write your pallas kernel here")


# speedup vs baseline: 3.0016x; 3.0016x over previous
import functools

import jax
import jax.numpy as jnp
from jax import lax
from jax.experimental import pallas as pl
from jax.experimental.pallas import tpu as pltpu
from jax.experimental.pallas import tpu_sc as plsc

N = 50000
E = 800000
H = 64
HE = 32
NC = 2
NS = 16
NW = NC * NS
CH = 3200
NJ = CH // 128
NQ = E // CH
CHS = 1280
NJS = CHS // 128
NQS = E // CHS
NP = 50176
NPH = NP // 2
NPHT = NPH // NS

f32 = jnp.float32
i32 = jnp.int32

@functools.lru_cache(maxsize=None)
def _mesh():
    return plsc.VectorSubcoreMesh(core_axis_name="c", subcore_axis_name="s",
                                  num_cores=NC, num_subcores=NS)


def _nchunks(wid):
    return (NQ - 1 - wid) // NW + 1


@functools.lru_cache(maxsize=None)
def _make_sc_gather():
    @functools.partial(
        pl.kernel,
        out_type=jax.ShapeDtypeStruct((E, HE), f32),
        mesh=_mesh(),
        scratch_types=[
            pltpu.VMEM((NJ, 128), i32),
            pltpu.VMEM((NJ, 128), i32),
            pltpu.VMEM((CH, HE), f32),
            pltpu.SemaphoreType.DMA,
        ],
        compiler_params=pltpu.CompilerParams(use_tc_tiling_on_sc=False))
    def body(pa_h, pb_h, src_h, dst_h, u_h, sidx, didx, u_v, sem):
        c = lax.axis_index("c")
        s = lax.axis_index("s")
        wid = c * NS + s

        def chunk(t, carry):
            q = wid + NW * t
            pltpu.sync_copy(src_h.at[q], sidx)
            pltpu.sync_copy(dst_h.at[q], didx)
            cps = [pltpu.async_copy(pa_h.at[sidx.at[j]],
                                    u_v.at[pl.ds(j * 128, 128)], sem)
                   for j in range(NJ)]
            for cp in cps:
                cp.wait()
            cps = [pltpu.async_copy(pb_h.at[didx.at[j]],
                                    u_v.at[pl.ds(j * 128, 128)], sem,
                                    add=True)
                   for j in range(NJ)]
            for cp in cps:
                cp.wait()
            pltpu.sync_copy(u_v, u_h.at[pl.ds(q * CH, CH)])
            return carry

        lax.fori_loop(0, _nchunks(wid), chunk, 0)

    return body


def _sc_gather(*args):
    return _make_sc_gather()(*args)


@functools.lru_cache(maxsize=None)
def _make_sc_count():
    @functools.partial(
        pl.kernel,
        out_type=jax.ShapeDtypeStruct((NW, NP), f32),
        mesh=_mesh(),
        scratch_types=[
            pltpu.VMEM((NJ, 128), i32),
            pltpu.VMEM((NP,), f32),
        ],
        compiler_params=pltpu.CompilerParams(use_tc_tiling_on_sc=False,
                                             needs_layout_passes=False))
    def body(dst_h, out_h, didx, ctab):
        c = lax.axis_index("c")
        s = lax.axis_index("s")
        wid = c * NS + s
        zero = jnp.zeros((16,), f32)
        one = jnp.ones((16,), f32)

        def zbody(i, carry):
            ctab[pl.ds(i * 16, 16)] = zero
            return carry

        lax.fori_loop(0, NP // 16, zbody, 0)

        def chunk(t, carry):
            q = wid + NW * t
            pltpu.sync_copy(dst_h.at[q], didx)
            for j in range(NJ):
                for k in range(8):
                    idx = didx[j, pl.ds(k * 16, 16)]
                    plsc.addupdate_scatter(ctab, [idx], one)
            return carry

        lax.fori_loop(0, _nchunks(wid), chunk, 0)
        pltpu.sync_copy(ctab, out_h.at[wid])

    return body


def _sc_count(*args):
    return _make_sc_count()(*args)


@functools.lru_cache(maxsize=None)
def _make_sc_scatter():
    return functools.partial(
        pl.kernel,
        out_type=jax.ShapeDtypeStruct((NP, HE), f32),
        mesh=_mesh(),
        scratch_types=[
            pltpu.VMEM((NJS, 128), i32),
            pltpu.VMEM((CHS, HE), f32),
            pltpu.VMEM_SHARED((NPH + 8, HE), f32),
            pltpu.SemaphoreType.DMA,
        ],
        compiler_params=pltpu.CompilerParams(use_tc_tiling_on_sc=False),
    )(_sc_scatter_body)


def _sc_scatter(*args):
    return _make_sc_scatter()(*args)


def _sc_scatter_body(lidx_h, val_h, z_h, out_h, didx, val_v, tab, sem):
    c = lax.axis_index("c")
    s = lax.axis_index("s")
    pltpu.sync_copy(z_h.at[pl.ds(s * NPHT, NPHT)],
                    tab.at[pl.ds(s * NPHT, NPHT)])
    plsc.subcore_barrier()

    def chunk(t, carry):
        q = s + NS * t
        pltpu.sync_copy(lidx_h.at[c, q], didx)
        pltpu.sync_copy(val_h.at[pl.ds(q * CHS, CHS)], val_v)
        cps = [pltpu.async_copy(val_v.at[pl.ds(j * 128, 128)],
                                tab.at[didx.at[j]], sem, add=True)
               for j in range(NJS)]
        for cp in cps:
            cp.wait()
        return carry

    lax.fori_loop(0, (NQS - 1 - s) // NS + 1, chunk, 0)
    plsc.subcore_barrier()
    pltpu.sync_copy(tab.at[pl.ds(s * NPHT, NPHT)],
                    out_h.at[pl.ds(c * NPH + s * NPHT, NPHT)])


def _gelu(x):
    return 0.5 * x * (1.0 + lax.erf(x * 0.7071067811865476))


def _lnp(x, J, g, b):
    m = jnp.dot(x, J, preferred_element_type=f32)
    xc = x - m
    v = jnp.dot(xc * xc, J, preferred_element_type=f32)
    return xc * lax.rsqrt(v + 1e-5) * g + b


def _mm(x, w):
    return jnp.dot(x, w, preferred_element_type=f32)


def _gru(x, h, wir, whr, br, wiz, whz, bz, win, whn, bi_n, bh_n):
    r = jax.nn.sigmoid(_mm(x, wir) + _mm(h, whr) + br)
    z = jax.nn.sigmoid(_mm(x, wiz) + _mm(h, whz) + bz)
    n = jnp.tanh(_mm(x, win) + bi_n + r * (_mm(h, whn) + bh_n))
    return (1.0 - z) * n + z * h


def _bd(w, k):
    return jnp.kron(jnp.eye(k, dtype=w.dtype), w)


def _tl(b, k):
    return jnp.tile(b, k)[None, :]


def _row_spec(rows, width, R):
    return pl.BlockSpec((R, width), lambda i: (i, 0))


def _w_spec(a):
    nd = a.ndim
    return pl.BlockSpec(a.shape, lambda i: (0,) * nd)


def _tc_call(body, R, row_ins, w_ins, out_widths, nrows, extra_out_specs=(),
             extra_out_shapes=()):
    grid = (nrows // R,)
    in_specs = ([_row_spec(nrows, a.shape[1], R) for a in row_ins]
                + [_w_spec(a) for a in w_ins])
    out_specs = [_row_spec(nrows, w, R) for w in out_widths]
    out_shape = [jax.ShapeDtypeStruct((nrows, w), f32) for w in out_widths]
    return pl.pallas_call(
        body, grid=grid, in_specs=in_specs,
        out_specs=list(out_specs) + list(extra_out_specs),
        out_shape=list(out_shape) + list(extra_out_shapes),
    )(*row_ins, *w_ins)


def kernel(node_state, edge_state, node_dyn, edge_dyn, node_static,
           edge_static, edge_index,
           W_ne, b_ne, g_ne, be_ne, W_ee, b_ee, g_ee, be_ee,
           W_m1, b_m1, W_m2, b_m2, W_mn, b_mn,
           W_m3, b_m3, W_m4, b_m4, W_mn2, b_mn2,
           W_nu, b_nu, W_eu, b_eu,
           Wih1, Whh1, bih1, bhh1, Wih2, Whh2, bih2, bhh2,
           Wihe, Whhe, bihe, bhhe, g_nn, b_nn, g_en, b_en):
    N2 = N // 2
    E4 = E // 4
    src3 = edge_index[0].reshape(NQ, NJ, 128)
    dst3 = edge_index[1].reshape(NQ, NJ, 128)

    ncat2 = jnp.concatenate([node_dyn, node_static], axis=1).reshape(N2, 48)
    nstate2 = node_state.reshape(N2, 2 * H)
    ed4 = edge_dyn.reshape(E4, 16)
    es4 = edge_static.reshape(E4, 32)
    est4 = edge_state.reshape(E4, 128)

    J32 = _bd(jnp.full((32, 32), 1.0 / 32, f32), 4)
    J64 = _bd(jnp.full((64, 64), 1.0 / 64, f32), 2)
    Mb = jnp.zeros((2, 64), f32).at[0, 0:32].set(1.0).at[1, 32:64].set(1.0)

    Wne2 = _bd(W_ne, 2)
    bne2, gne2, bene2 = _tl(b_ne, 2), _tl(g_ne, 2), _tl(be_ne, 2)
    A1_2, B1_2 = _bd(W_m1[:H], 2), _bd(W_m1[H:2 * H], 2)
    A3_2, B3_2 = _bd(W_m3[:H], 2), _bd(W_m3[H:2 * H], 2)

    WeeD4, WeeS4 = _bd(W_ee[:4], 4), _bd(W_ee[4:], 4)
    bee4, gee4, bee_ln4 = _tl(b_ee, 4), _tl(g_ee, 4), _tl(be_ee, 4)
    C1_4, bm1_4 = _bd(W_m1[2 * H:], 4), _tl(b_m1, 4)
    C3_4, bm3_4 = _bd(W_m3[2 * H:], 4), _tl(b_m3, 4)
    Wm2_4, bm2_4 = _bd(W_m2, 4), _tl(b_m2, 4)
    Wm4_4, bm4_4 = _bd(W_m4, 4), _tl(b_m4, 4)
    WeuT4, WeuB4, beu4 = _bd(W_eu[:HE], 4), _bd(W_eu[HE:], 4), _tl(b_eu, 4)
    gen4, ben4 = _tl(g_en, 4), _tl(b_en, 4)

    def gru_w(Wih, Whh, bih, bhh, hh, k):
        out = []
        for j in range(3):
            out += [_bd(Wih[:, j * hh:(j + 1) * hh], k),
                    _bd(Whh[:, j * hh:(j + 1) * hh], k)]
        out += [_tl(bih[0:hh] + bhh[0:hh], k),
                _tl(bih[hh:2 * hh] + bhh[hh:2 * hh], k),
                _tl(bih[2 * hh:], k), _tl(bhh[2 * hh:], k)]
        return out

    grue = gru_w(Wihe, Whhe, bihe, bhhe, HE, 4)
    gru1 = gru_w(Wih1, Whh1, bih1, bhh1, H, 2)
    gru2 = gru_w(Wih2, Whh2, bih2, bhh2, H, 2)

    Wmn2k, bmn2k = _bd(W_mn, 2), _tl(b_mn, 2)
    Wmn2_2, bmn2_2 = _bd(W_mn2, 2), _tl(b_mn2, 2)
    WnuT2, WnuB2, bnu2 = _bd(W_nu[:H], 2), _bd(W_nu[H:], 2), _tl(b_nu, 2)
    gnn2, bnn2 = _tl(g_nn, 2), _tl(b_nn, 2)

    ztab = jnp.zeros((NPH, HE), f32)

    RA = 1000

    def tca(ncat_ref, nst_ref, wne, bne, gne, bene, j64, a1, b1,
            nb_ref, pa_ref, pb_ref, num_ref):
        x = ncat_ref[...]
        pre = _mm(x, wne[...]) + bne[...]
        nb = _gelu(_lnp(pre, j64[...], gne[...], bene[...])) + nst_ref[...]
        nb_ref[...] = nb
        pa_ref[...] = _mm(nb, a1[...])
        pb_ref[...] = _mm(nb, b1[...])
        lix = lax.broadcasted_iota(i32, x.shape, 1)
        is_last = (lix == 23) | (lix == 47)
        cnt = jnp.sum(jnp.where(is_last & (x < 0.5), 1, 0).astype(i32))

        @pl.when(pl.program_id(0) == 0)
        def _():
            num_ref[0, 0] = 0

        num_ref[0, 0] += cnt

    nb2, pa2d, pb2d, num1d = _tc_call(
        tca, RA, [ncat2, nstate2],
        [Wne2, bne2, gne2, bene2, J64, A1_2, B1_2],
        [128, 64, 64], N2,
        extra_out_specs=[pl.BlockSpec((1, 1), lambda i: (0, 0),
                                      memory_space=pltpu.SMEM)],
        extra_out_shapes=[jax.ShapeDtypeStruct((1, 1), i32)],
    )

    u1 = _sc_gather(pa2d.reshape(N, HE), pb2d.reshape(N, HE), src3, dst3)
    cnt32 = _sc_count(dst3)

    def tcd0(c_ref, o_ref):
        o_ref[...] = jnp.sum(c_ref[...], axis=0, keepdims=True)

    cnt1 = pl.pallas_call(
        tcd0, grid=(NP // 1024,),
        in_specs=[pl.BlockSpec((NW, 1024), lambda i: (0, i))],
        out_specs=[pl.BlockSpec((1, 1024), lambda i: (0, i))],
        out_shape=[jax.ShapeDtypeStruct((1, NP), f32)],
    )(cnt32)[0]
    cntp = cnt1[0, :N].reshape(N2, 2)

    RC = 1000

    def tcc(ed_ref, es_ref, est_ref, u1_ref,
            weeD, weeS, bee, gee, beeln, j32, c1, bm1, wm2, bm2, c3, bm3,
            weuT, weuB, beu,
            wir, whr, wiz, whz, win, whn, br, bz, bi_n, bh_n,
            gen, ben,
            msg_ref, ec2_ref, ne_ref):
        est = est_ref[...]
        pre = _mm(ed_ref[...], weeD[...]) + _mm(es_ref[...], weeS[...]) + bee[...]
        eb = _gelu(_lnp(pre, j32[...], gee[...], beeln[...])) + est
        ec1 = _mm(eb, c1[...]) + bm1[...]
        g1 = _gelu(u1_ref[...] + ec1)
        msg = _mm(g1, wm2[...]) + bm2[...]
        msg_ref[...] = msg
        ec2_ref[...] = _mm(eb, c3[...]) + bm3[...]
        ein = _gelu(_mm(eb, weuT[...]) + _mm(msg, weuB[...]) + beu[...])
        ne = _gru(ein, est, wir[...], whr[...], br[...], wiz[...], whz[...],
                  bz[...], win[...], whn[...], bi_n[...], bh_n[...])
        ne_ref[...] = _lnp(ne, j32[...], gen[...], ben[...])

    msg4, ec2_4, nedge4 = _tc_call(
        tcc, RC, [ed4, es4, est4, u1.reshape(E4, 128)],
        [WeeD4, WeeS4, bee4, gee4, bee_ln4, J32, C1_4, bm1_4, Wm2_4, bm2_4,
         C3_4, bm3_4, WeuT4, WeuB4, beu4] + grue[:6] + grue[6:]
        + [gen4, ben4],
        [128, 128, 128], E4)

    def tcl(d_ref, o0_ref, o1_ref):
        d = d_ref[...]
        o0_ref[...] = jnp.where(d >= NPH, NPH, d)
        v = d - NPH
        o1_ref[...] = jnp.where(v < 0, NPH, v)

    lid0, lid1 = pl.pallas_call(
        tcl,
        out_shape=[jax.ShapeDtypeStruct((E // 128, 128), i32)] * 2,
    )(edge_index[1].reshape(E // 128, 128))
    lidx2 = jnp.stack([lid0, lid1]).reshape(2, NQS, NJS, 128)

    part1 = _sc_scatter(lidx2, msg4.reshape(E, HE), ztab)

    RD = 1000
    p1 = part1[:N].reshape(N2, 64)

    def tcd(s0_ref, c0_ref, nb_ref, mb, wmn, bmn, a3, b3,
            nh1_ref, pa2_ref, pb2_ref):
        deg = jnp.maximum(_mm(c0_ref[...], mb[...]), 1.0)
        agg = s0_ref[...] / deg
        nh1 = nb_ref[...] + _gelu(_mm(agg, wmn[...]) + bmn[...])
        nh1_ref[...] = nh1
        pa2_ref[...] = _mm(nh1, a3[...])
        pb2_ref[...] = _mm(nh1, b3[...])

    nh1, pa2_2, pb2_2 = _tc_call(
        tcd, RD, [p1, cntp, nb2],
        [Mb, Wmn2k, bmn2k, A3_2, B3_2],
        [128, 64, 64], N2)

    u2 = _sc_gather(pa2_2.reshape(N, HE), pb2_2.reshape(N, HE), src3, dst3)

    RE = 1000

    def tce(u2_ref, ec2_ref, wm4, bm4, msg2_ref):
        g2 = _gelu(u2_ref[...] + ec2_ref[...])
        msg2_ref[...] = _mm(g2, wm4[...]) + bm4[...]

    (msg2_4,) = _tc_call(tce, RE, [u2.reshape(E4, 128), ec2_4],
                         [Wm4_4, bm4_4], [128], E4)

    part2 = _sc_scatter(lidx2, msg2_4.reshape(E, HE), ztab)

    RF = 1000
    p2 = part2[:N].reshape(N2, 64)

    def tcf(s0_ref, c0_ref, nh1_ref, nst_ref, num_ref,
            mb, wmn2, bmn2, wnuT, wnuB, bnu,
            wir1, whr1, wiz1, whz1, win1, whn1, br1, bz1, bin1, bhn1,
            wir2, whr2, wiz2, whz2, win2, whn2, br2, bz2, bin2, bhn2,
            j64, gnn, bnn,
            out_ref):
        deg = jnp.maximum(_mm(c0_ref[...], mb[...]), 1.0)
        agg2 = s0_ref[...] / deg
        q = _gelu(_mm(agg2, wmn2[...]) + bmn2[...])
        nh1 = nh1_ref[...]
        nin = _gelu(_mm(nh1, wnuT[...]) + _mm(q, wnuB[...]) + bnu[...])
        h = nst_ref[...]
        o1 = _gru(nin, h, wir1[...], whr1[...], br1[...], wiz1[...],
                  whz1[...], bz1[...], win1[...], whn1[...], bin1[...],
                  bhn1[...])
        o2 = _gru(nin, h, wir2[...], whr2[...], br2[...], wiz2[...],
                  whz2[...], bz2[...], win2[...], whn2[...], bin2[...],
                  bhn2[...])
        rix = lax.broadcasted_iota(i32, o1.shape, 0)
        lix = lax.broadcasted_iota(i32, o1.shape, 1)
        nid = ((pl.program_id(0) * RF + rix) * 2
               + jnp.where(lix >= H, 1, 0))
        o = jnp.where(nid < num_ref[0, 0], o1, o2)
        out_ref[...] = _lnp(o, j64[...], gnn[...], bnn[...])

    grid = (N2 // RF,)
    in_specs = ([_row_spec(N2, 64, RF)] + [_row_spec(N2, 2, RF)]
                + [_row_spec(N2, 128, RF)] * 2
                + [pl.BlockSpec((1, 1), lambda i: (0, 0),
                                memory_space=pltpu.SMEM)]
                + [_w_spec(a) for a in
                   [Mb, Wmn2_2, bmn2_2, WnuT2, WnuB2, bnu2] + gru1 + gru2
                   + [J64, gnn2, bnn2]])
    next_node2 = pl.pallas_call(
        tcf, grid=grid, in_specs=in_specs,
        out_specs=[_row_spec(N2, 128, RF)],
        out_shape=[jax.ShapeDtypeStruct((N2, 128), f32)],
    )(p2, cntp, nh1, nstate2, num1d,
      Mb, Wmn2_2, bmn2_2, WnuT2, WnuB2, bnu2, *gru1, *gru2,
      J64, gnn2, bnn2)[0]

    return next_node2.reshape(N, H), nedge4.reshape(E, HE)


# speedup vs baseline: 3.1710x; 1.0564x over previous
import functools

import jax
import jax.numpy as jnp
from jax import lax
from jax.experimental import pallas as pl
from jax.experimental.pallas import tpu as pltpu
from jax.experimental.pallas import tpu_sc as plsc

N = 50000
E = 800000
H = 64
HE = 32
NC = 2
NS = 16
NW = NC * NS
CH = 3200
NJ = CH // 128
NQ = E // CH
CHS = 1280
NJS = CHS // 128
NQS = E // CHS
NP = 50176
NPH = NP // 2
NPHT = NPH // NS

f32 = jnp.float32
i32 = jnp.int32

@functools.lru_cache(maxsize=None)
def _mesh():
    return plsc.VectorSubcoreMesh(core_axis_name="c", subcore_axis_name="s",
                                  num_cores=NC, num_subcores=NS)


def _nchunks(wid):
    return (NQ - 1 - wid) // NW + 1


@functools.lru_cache(maxsize=None)
def _make_sc_gather():
    @functools.partial(
        pl.kernel,
        out_type=jax.ShapeDtypeStruct((E, HE), f32),
        mesh=_mesh(),
        scratch_types=[
            pltpu.VMEM((NJ, 128), i32),
            pltpu.VMEM((NJ, 128), i32),
            pltpu.VMEM((CH, HE), f32),
            pltpu.SemaphoreType.DMA,
        ],
        compiler_params=pltpu.CompilerParams(use_tc_tiling_on_sc=False))
    def body(pa_h, pb_h, src_h, dst_h, u_h, sidx, didx, u_v, sem):
        c = lax.axis_index("c")
        s = lax.axis_index("s")
        wid = c * NS + s

        def chunk(t, carry):
            q = wid + NW * t
            pltpu.sync_copy(src_h.at[q], sidx)
            pltpu.sync_copy(dst_h.at[q], didx)
            cps = [pltpu.async_copy(pa_h.at[sidx.at[j]],
                                    u_v.at[pl.ds(j * 128, 128)], sem)
                   for j in range(NJ)]
            for cp in cps:
                cp.wait()
            cps = [pltpu.async_copy(pb_h.at[didx.at[j]],
                                    u_v.at[pl.ds(j * 128, 128)], sem,
                                    add=True)
                   for j in range(NJ)]
            for cp in cps:
                cp.wait()
            pltpu.sync_copy(u_v, u_h.at[pl.ds(q * CH, CH)])
            return carry

        lax.fori_loop(0, _nchunks(wid), chunk, 0)

    return body


def _sc_gather(*args):
    return _make_sc_gather()(*args)


@functools.lru_cache(maxsize=None)
def _make_sc_count():
    @functools.partial(
        pl.kernel,
        out_type=jax.ShapeDtypeStruct((NW, NP), f32),
        mesh=_mesh(),
        scratch_types=[
            pltpu.VMEM((NJ, 128), i32),
            pltpu.VMEM((NP,), f32),
        ],
        compiler_params=pltpu.CompilerParams(use_tc_tiling_on_sc=False,
                                             needs_layout_passes=False))
    def body(dst_h, out_h, didx, ctab):
        c = lax.axis_index("c")
        s = lax.axis_index("s")
        wid = c * NS + s
        zero = jnp.zeros((16,), f32)
        one = jnp.ones((16,), f32)

        def zbody(i, carry):
            ctab[pl.ds(i * 16, 16)] = zero
            return carry

        lax.fori_loop(0, NP // 16, zbody, 0)

        def chunk(t, carry):
            q = wid + NW * t
            pltpu.sync_copy(dst_h.at[q], didx)
            for j in range(NJ):
                for k in range(8):
                    idx = didx[j, pl.ds(k * 16, 16)]
                    plsc.addupdate_scatter(ctab, [idx], one)
            return carry

        lax.fori_loop(0, _nchunks(wid), chunk, 0)
        pltpu.sync_copy(ctab, out_h.at[wid])

    return body


def _sc_count(*args):
    return _make_sc_count()(*args)


@functools.lru_cache(maxsize=None)
def _make_sc_scatter():
    return functools.partial(
        pl.kernel,
        out_type=jax.ShapeDtypeStruct((NP, HE), f32),
        mesh=_mesh(),
        scratch_types=[
            pltpu.VMEM((NJS, 128), i32),
            pltpu.VMEM((CHS, HE), f32),
            pltpu.VMEM_SHARED((NPH + 8, HE), f32),
            pltpu.SemaphoreType.DMA,
        ],
        compiler_params=pltpu.CompilerParams(use_tc_tiling_on_sc=False),
    )(_sc_scatter_body)


def _sc_scatter(*args):
    return _make_sc_scatter()(*args)


def _sc_scatter_body(lidx_h, val_h, z_h, out_h, didx, val_v, tab, sem):
    c = lax.axis_index("c")
    s = lax.axis_index("s")
    pltpu.sync_copy(z_h.at[pl.ds(s * NPHT, NPHT)],
                    tab.at[pl.ds(s * NPHT, NPHT)])
    plsc.subcore_barrier()

    def chunk(t, carry):
        q = s + NS * t
        pltpu.sync_copy(lidx_h.at[c, q], didx)
        pltpu.sync_copy(val_h.at[pl.ds(q * CHS, CHS)], val_v)
        cps = [pltpu.async_copy(val_v.at[pl.ds(j * 128, 128)],
                                tab.at[didx.at[j]], sem, add=True)
               for j in range(NJS)]
        for cp in cps:
            cp.wait()
        return carry

    lax.fori_loop(0, (NQS - 1 - s) // NS + 1, chunk, 0)
    plsc.subcore_barrier()
    pltpu.sync_copy(tab.at[pl.ds(s * NPHT, NPHT)],
                    out_h.at[pl.ds(c * NPH + s * NPHT, NPHT)])


def _gelu(x):
    return 0.5 * x * (1.0 + lax.erf(x * 0.7071067811865476))


def _lnp(x, J, g, b):
    m = jnp.dot(x, J, preferred_element_type=f32)
    xc = x - m
    v = jnp.dot(xc * xc, J, preferred_element_type=f32)
    return xc * lax.rsqrt(v + 1e-5) * g + b


def _mm(x, w):
    return jnp.dot(x, w, preferred_element_type=f32)


def _gru(x, h, wir, whr, br, wiz, whz, bz, win, whn, bi_n, bh_n):
    r = jax.nn.sigmoid(_mm(x, wir) + _mm(h, whr) + br)
    z = jax.nn.sigmoid(_mm(x, wiz) + _mm(h, whz) + bz)
    n = jnp.tanh(_mm(x, win) + bi_n + r * (_mm(h, whn) + bh_n))
    return (1.0 - z) * n + z * h


def _bd(w, k):
    return jnp.kron(jnp.eye(k, dtype=w.dtype), w)


def _tl(b, k):
    return jnp.tile(b, k)[None, :]


def _row_spec(rows, width, R):
    return pl.BlockSpec((R, width), lambda i: (i, 0))


def _w_spec(a):
    nd = a.ndim
    return pl.BlockSpec(a.shape, lambda i: (0,) * nd)


def _tc_call(body, R, row_ins, w_ins, out_widths, nrows, extra_out_specs=(),
             extra_out_shapes=()):
    grid = (nrows // R,)
    in_specs = ([_row_spec(nrows, a.shape[1], R) for a in row_ins]
                + [_w_spec(a) for a in w_ins])
    out_specs = [_row_spec(nrows, w, R) for w in out_widths]
    out_shape = [jax.ShapeDtypeStruct((nrows, w), f32) for w in out_widths]
    return pl.pallas_call(
        body, grid=grid, in_specs=in_specs,
        out_specs=list(out_specs) + list(extra_out_specs),
        out_shape=list(out_shape) + list(extra_out_shapes),
    )(*row_ins, *w_ins)


def kernel(node_state, edge_state, node_dyn, edge_dyn, node_static,
           edge_static, edge_index,
           W_ne, b_ne, g_ne, be_ne, W_ee, b_ee, g_ee, be_ee,
           W_m1, b_m1, W_m2, b_m2, W_mn, b_mn,
           W_m3, b_m3, W_m4, b_m4, W_mn2, b_mn2,
           W_nu, b_nu, W_eu, b_eu,
           Wih1, Whh1, bih1, bhh1, Wih2, Whh2, bih2, bhh2,
           Wihe, Whhe, bihe, bhhe, g_nn, b_nn, g_en, b_en):
    N2 = N // 2
    E4 = E // 4
    src3 = edge_index[0].reshape(NQ, NJ, 128)
    dst3 = edge_index[1].reshape(NQ, NJ, 128)

    ncat2 = jnp.concatenate([node_dyn, node_static], axis=1).reshape(N2, 48)
    nstate2 = node_state.reshape(N2, 2 * H)
    edt = edge_dyn.T
    es8 = edge_static.T
    estT = edge_state.T

    J32 = _bd(jnp.full((32, 32), 1.0 / 32, f32), 4)
    J64 = _bd(jnp.full((64, 64), 1.0 / 64, f32), 2)
    Mb = jnp.zeros((2, 64), f32).at[0, 0:32].set(1.0).at[1, 32:64].set(1.0)

    Wne2 = _bd(W_ne, 2)
    bne2, gne2, bene2 = _tl(b_ne, 2), _tl(g_ne, 2), _tl(be_ne, 2)
    A1_2, B1_2 = _bd(W_m1[:H], 2), _bd(W_m1[H:2 * H], 2)
    A3_2, B3_2 = _bd(W_m3[:H], 2), _bd(W_m3[H:2 * H], 2)

    bee4, gee4, bee_ln4 = _tl(b_ee, 4), _tl(g_ee, 4), _tl(be_ee, 4)
    C1_4, bm1_4 = _bd(W_m1[2 * H:], 4), _tl(b_m1, 4)
    C3_4, bm3_4 = _bd(W_m3[2 * H:], 4), _tl(b_m3, 4)
    Wm2_4, bm2_4 = _bd(W_m2, 4), _tl(b_m2, 4)
    Wm4_4, bm4_4 = _bd(W_m4, 4), _tl(b_m4, 4)
    WeuT4, WeuB4, beu4 = _bd(W_eu[:HE], 4), _bd(W_eu[HE:], 4), _tl(b_eu, 4)
    gen4, ben4 = _tl(g_en, 4), _tl(b_en, 4)

    def gru_w(Wih, Whh, bih, bhh, hh, k):
        out = []
        for j in range(3):
            out += [_bd(Wih[:, j * hh:(j + 1) * hh], k),
                    _bd(Whh[:, j * hh:(j + 1) * hh], k)]
        out += [_tl(bih[0:hh] + bhh[0:hh], k),
                _tl(bih[hh:2 * hh] + bhh[hh:2 * hh], k),
                _tl(bih[2 * hh:], k), _tl(bhh[2 * hh:], k)]
        return out

    grue = gru_w(Wihe, Whhe, bihe, bhhe, HE, 4)
    gru1 = gru_w(Wih1, Whh1, bih1, bhh1, H, 2)
    gru2 = gru_w(Wih2, Whh2, bih2, bhh2, H, 2)

    Wmn2k, bmn2k = _bd(W_mn, 2), _tl(b_mn, 2)
    Wmn2_2, bmn2_2 = _bd(W_mn2, 2), _tl(b_mn2, 2)
    WnuT2, WnuB2, bnu2 = _bd(W_nu[:H], 2), _bd(W_nu[H:], 2), _tl(b_nu, 2)
    gnn2, bnn2 = _tl(g_nn, 2), _tl(b_nn, 2)

    ztab = jnp.zeros((NPH, HE), f32)

    RA = 1000

    def tca(ncat_ref, nst_ref, wne, bne, gne, bene, j64, a1, b1,
            nb_ref, pa_ref, pb_ref, num_ref):
        x = ncat_ref[...]
        pre = _mm(x, wne[...]) + bne[...]
        nb = _gelu(_lnp(pre, j64[...], gne[...], bene[...])) + nst_ref[...]
        nb_ref[...] = nb
        pa_ref[...] = _mm(nb, a1[...])
        pb_ref[...] = _mm(nb, b1[...])
        lix = lax.broadcasted_iota(i32, x.shape, 1)
        is_last = (lix == 23) | (lix == 47)
        cnt = jnp.sum(jnp.where(is_last & (x < 0.5), 1, 0).astype(i32))

        @pl.when(pl.program_id(0) == 0)
        def _():
            num_ref[0, 0] = 0

        num_ref[0, 0] += cnt

    nb2, pa2d, pb2d, num1d = _tc_call(
        tca, RA, [ncat2, nstate2],
        [Wne2, bne2, gne2, bene2, J64, A1_2, B1_2],
        [128, 64, 64], N2,
        extra_out_specs=[pl.BlockSpec((1, 1), lambda i: (0, 0),
                                      memory_space=pltpu.SMEM)],
        extra_out_shapes=[jax.ShapeDtypeStruct((1, 1), i32)],
    )

    u1 = _sc_gather(pa2d.reshape(N, HE), pb2d.reshape(N, HE), src3, dst3)
    cnt32 = _sc_count(dst3)

    def tcd0(c_ref, o_ref):
        o_ref[...] = jnp.sum(c_ref[...], axis=0, keepdims=True)

    cnt1 = pl.pallas_call(
        tcd0, grid=(NP // 1024,),
        in_specs=[pl.BlockSpec((NW, 1024), lambda i: (0, i))],
        out_specs=[pl.BlockSpec((1, 1024), lambda i: (0, i))],
        out_shape=[jax.ShapeDtypeStruct((1, NP), f32)],
    )(cnt32)[0]
    cntp = cnt1[0, :N].reshape(N2, 2)

    RC = 1600
    EC = 4 * RC

    def _pack(x):
        x3 = x.reshape(RC, 4, HE)
        return jnp.concatenate([x3[:, p, :] for p in range(4)], axis=1)

    def _unpack(y):
        parts = [y[:, p * HE:(p + 1) * HE] for p in range(4)]
        return jnp.stack(parts, axis=1).reshape(EC, HE)

    def tcc(ed_ref, es_ref, est_ref, u1_ref,
            wd, ws, bee, gee, beeln, j32, c1, bm1, wm2, bm2, c3, bm3,
            weuT, weuB, beu,
            wir, whr, wiz, whz, win, whn, br, bz, bi_n, bh_n,
            gen, ben,
            msg_ref, ec2_ref, ne_ref):
        est = _pack(est_ref[...].T)
        pre32 = (_mm(ed_ref[...].T, wd[...]) + _mm(es_ref[...].T, ws[...]))
        pre = _pack(pre32) + bee[...]
        eb = _gelu(_lnp(pre, j32[...], gee[...], beeln[...])) + est
        ec1 = _mm(eb, c1[...]) + bm1[...]
        g1 = _gelu(u1_ref[...] + ec1)
        msg = _mm(g1, wm2[...]) + bm2[...]
        msg_ref[...] = msg
        ec2_ref[...] = _mm(eb, c3[...]) + bm3[...]
        ein = _gelu(_mm(eb, weuT[...]) + _mm(msg, weuB[...]) + beu[...])
        ne = _gru(ein, est, wir[...], whr[...], br[...], wiz[...], whz[...],
                  bz[...], win[...], whn[...], bi_n[...], bh_n[...])
        ne_ref[...] = _unpack(_lnp(ne, j32[...], gen[...], ben[...])).T

    w_ins = ([W_ee[:4], W_ee[4:], bee4, gee4, bee_ln4, J32, C1_4, bm1_4,
              Wm2_4, bm2_4, C3_4, bm3_4, WeuT4, WeuB4, beu4]
             + grue[:6] + grue[6:] + [gen4, ben4])
    msg4, ec2_4, nedgeT = pl.pallas_call(
        tcc, grid=(E4 // RC,),
        in_specs=([pl.BlockSpec((4, EC), lambda i: (0, i)),
                   pl.BlockSpec((8, EC), lambda i: (0, i)),
                   pl.BlockSpec((32, EC), lambda i: (0, i)),
                   _row_spec(E4, 128, RC)]
                  + [_w_spec(a) for a in w_ins]),
        out_specs=[_row_spec(E4, 128, RC), _row_spec(E4, 128, RC),
                   pl.BlockSpec((32, EC), lambda i: (0, i))],
        out_shape=[jax.ShapeDtypeStruct((E4, 128), f32),
                   jax.ShapeDtypeStruct((E4, 128), f32),
                   jax.ShapeDtypeStruct((HE, E), f32)],
    )(edt, es8, estT, u1.reshape(E4, 128), *w_ins)

    def tcl(d_ref, o0_ref, o1_ref):
        d = d_ref[...]
        o0_ref[...] = jnp.where(d >= NPH, NPH, d)
        v = d - NPH
        o1_ref[...] = jnp.where(v < 0, NPH, v)

    lid0, lid1 = pl.pallas_call(
        tcl,
        out_shape=[jax.ShapeDtypeStruct((E // 128, 128), i32)] * 2,
    )(edge_index[1].reshape(E // 128, 128))
    lidx2 = jnp.stack([lid0, lid1]).reshape(2, NQS, NJS, 128)

    part1 = _sc_scatter(lidx2, msg4.reshape(E, HE), ztab)

    RD = 1000
    p1 = part1[:N].reshape(N2, 64)

    def tcd(s0_ref, c0_ref, nb_ref, mb, wmn, bmn, a3, b3,
            nh1_ref, pa2_ref, pb2_ref):
        deg = jnp.maximum(_mm(c0_ref[...], mb[...]), 1.0)
        agg = s0_ref[...] / deg
        nh1 = nb_ref[...] + _gelu(_mm(agg, wmn[...]) + bmn[...])
        nh1_ref[...] = nh1
        pa2_ref[...] = _mm(nh1, a3[...])
        pb2_ref[...] = _mm(nh1, b3[...])

    nh1, pa2_2, pb2_2 = _tc_call(
        tcd, RD, [p1, cntp, nb2],
        [Mb, Wmn2k, bmn2k, A3_2, B3_2],
        [128, 64, 64], N2)

    u2 = _sc_gather(pa2_2.reshape(N, HE), pb2_2.reshape(N, HE), src3, dst3)

    RE = 1000

    def tce(u2_ref, ec2_ref, wm4, bm4, msg2_ref):
        g2 = _gelu(u2_ref[...] + ec2_ref[...])
        msg2_ref[...] = _mm(g2, wm4[...]) + bm4[...]

    (msg2_4,) = _tc_call(tce, RE, [u2.reshape(E4, 128), ec2_4],
                         [Wm4_4, bm4_4], [128], E4)

    part2 = _sc_scatter(lidx2, msg2_4.reshape(E, HE), ztab)

    RF = 1000
    p2 = part2[:N].reshape(N2, 64)

    def tcf(s0_ref, c0_ref, nh1_ref, nst_ref, num_ref,
            mb, wmn2, bmn2, wnuT, wnuB, bnu,
            wir1, whr1, wiz1, whz1, win1, whn1, br1, bz1, bin1, bhn1,
            wir2, whr2, wiz2, whz2, win2, whn2, br2, bz2, bin2, bhn2,
            j64, gnn, bnn,
            out_ref):
        deg = jnp.maximum(_mm(c0_ref[...], mb[...]), 1.0)
        agg2 = s0_ref[...] / deg
        q = _gelu(_mm(agg2, wmn2[...]) + bmn2[...])
        nh1 = nh1_ref[...]
        nin = _gelu(_mm(nh1, wnuT[...]) + _mm(q, wnuB[...]) + bnu[...])
        h = nst_ref[...]
        o1 = _gru(nin, h, wir1[...], whr1[...], br1[...], wiz1[...],
                  whz1[...], bz1[...], win1[...], whn1[...], bin1[...],
                  bhn1[...])
        o2 = _gru(nin, h, wir2[...], whr2[...], br2[...], wiz2[...],
                  whz2[...], bz2[...], win2[...], whn2[...], bin2[...],
                  bhn2[...])
        rix = lax.broadcasted_iota(i32, o1.shape, 0)
        lix = lax.broadcasted_iota(i32, o1.shape, 1)
        nid = ((pl.program_id(0) * RF + rix) * 2
               + jnp.where(lix >= H, 1, 0))
        o = jnp.where(nid < num_ref[0, 0], o1, o2)
        out_ref[...] = _lnp(o, j64[...], gnn[...], bnn[...])

    grid = (N2 // RF,)
    in_specs = ([_row_spec(N2, 64, RF)] + [_row_spec(N2, 2, RF)]
                + [_row_spec(N2, 128, RF)] * 2
                + [pl.BlockSpec((1, 1), lambda i: (0, 0),
                                memory_space=pltpu.SMEM)]
                + [_w_spec(a) for a in
                   [Mb, Wmn2_2, bmn2_2, WnuT2, WnuB2, bnu2] + gru1 + gru2
                   + [J64, gnn2, bnn2]])
    next_node2 = pl.pallas_call(
        tcf, grid=grid, in_specs=in_specs,
        out_specs=[_row_spec(N2, 128, RF)],
        out_shape=[jax.ShapeDtypeStruct((N2, 128), f32)],
    )(p2, cntp, nh1, nstate2, num1d,
      Mb, Wmn2_2, bmn2_2, WnuT2, WnuB2, bnu2, *gru1, *gru2,
      J64, gnn2, bnn2)[0]

    return next_node2.reshape(N, H), nedgeT.T
